# Initial kernel scaffold; baseline (speedup 1.0000x reference)
import jax
import jax.numpy as jnp
from jax.experimental import pallas as pl


def kernel(weights):
    raise NotImplementedError("write your pallas kernel here")


# speedup vs baseline: 1.1685x; 1.1685x over previous
import jax
import jax.numpy as jnp
from jax.experimental import pallas as pl
from jax.experimental.pallas import tpu as pltpu

N = 1000000
ROWS, COLS = 1024, 1024
PAD = ROWS * COLS
BLOCK_ROWS = 128
GRID = ROWS // BLOCK_ROWS

_KS0 = 0
_KS1 = 42
_KS2 = _KS0 ^ _KS1 ^ 0x1BD11BDA
_ROT = ((13, 15, 26, 6), (17, 29, 16, 24))


def _rotl(x, d):
    return (x << jnp.uint32(d)) | (x >> jnp.uint32(32 - d))


def _threefry_bits(c2):
    ks = (jnp.uint32(_KS0), jnp.uint32(_KS1), jnp.uint32(_KS2))
    x0 = jnp.full(c2.shape, _KS0, jnp.uint32)
    x1 = c2 + ks[1]

    def rounds(x0, x1, rs):
        for r in rs:
            x0 = x0 + x1
            x1 = _rotl(x1, r)
            x1 = x0 ^ x1
        return x0, x1

    x0, x1 = rounds(x0, x1, _ROT[0])
    x0, x1 = x0 + ks[1], x1 + ks[2] + jnp.uint32(1)
    x0, x1 = rounds(x0, x1, _ROT[1])
    x0, x1 = x0 + ks[2], x1 + ks[0] + jnp.uint32(2)
    x0, x1 = rounds(x0, x1, _ROT[0])
    x0, x1 = x0 + ks[0], x1 + ks[1] + jnp.uint32(3)
    x0, x1 = rounds(x0, x1, _ROT[1])
    x0, x1 = x0 + ks[1], x1 + ks[2] + jnp.uint32(4)
    x0, x1 = rounds(x0, x1, _ROT[0])
    x0, x1 = x0 + ks[2], x1 + ks[0] + jnp.uint32(5)
    return x0 ^ x1


def _body(w_ref, out_ref, best_val, best_idx):
    j = pl.program_id(0)

    @pl.when(j == 0)
    def _():
        best_val[0] = -jnp.inf
        best_idx[0] = jnp.int32(2**31 - 1)

    row = jax.lax.broadcasted_iota(jnp.int32, (BLOCK_ROWS, COLS), 0)
    col = jax.lax.broadcasted_iota(jnp.int32, (BLOCK_ROWS, COLS), 1)
    pos = (j * BLOCK_ROWS + row) * COLS + col

    bits = _threefry_bits(pos.astype(jnp.uint32))
    fbits = (bits >> jnp.uint32(9)) | jnp.uint32(0x3F800000)
    f = jax.lax.bitcast_convert_type(fbits, jnp.float32) - jnp.float32(1.0)
    tiny = jnp.float32(1.1754943508222875e-38)
    u = jnp.maximum(tiny, f * (jnp.float32(1.0) - tiny) + tiny)
    g = -jnp.log(-jnp.log(u))
    z = g + jnp.log(w_ref[...])
    z = jnp.where(pos < N, z, -jnp.inf)

    m = jnp.max(z)
    bi = jnp.min(jnp.where(z == m, pos, jnp.int32(2**31 - 1)))
    better = (m > best_val[0]) | ((m == best_val[0]) & (bi < best_idx[0]))
    best_val[0] = jnp.where(better, m, best_val[0])
    best_idx[0] = jnp.where(better, bi, best_idx[0])

    @pl.when(j == GRID - 1)
    def _():
        out_ref[0] = best_idx[0]


def kernel(weights):
    wp = jnp.pad(weights, (0, PAD - N)).reshape(ROWS, COLS)
    idx = pl.pallas_call(
        _body,
        grid=(GRID,),
        in_specs=[pl.BlockSpec((BLOCK_ROWS, COLS), lambda j: (j, 0))],
        out_specs=pl.BlockSpec(memory_space=pltpu.SMEM),
        out_shape=jax.ShapeDtypeStruct((1,), jnp.int32),
        scratch_shapes=[
            pltpu.SMEM((1,), jnp.float32),
            pltpu.SMEM((1,), jnp.int32),
        ],
    )(wp)
    return idx


# speedup vs baseline: 1.5933x; 1.3636x over previous
import jax
import jax.numpy as jnp
from jax.experimental import pallas as pl
from jax.experimental.pallas import tpu as pltpu

N = 1000000
ROWS, COLS = 1024, 1024
PAD = ROWS * COLS
BLOCK_ROWS = 128
GRID = ROWS // BLOCK_ROWS

_KS0 = 0
_KS1 = 42
_KS2 = _KS0 ^ _KS1 ^ 0x1BD11BDA
_ROT = ((13, 15, 26, 6), (17, 29, 16, 24))


def _rotl(x, d):
    return (x << jnp.uint32(d)) | (x >> jnp.uint32(32 - d))


def _threefry_bits(c2):
    ks = (jnp.uint32(_KS0), jnp.uint32(_KS1), jnp.uint32(_KS2))
    x0 = jnp.full(c2.shape, _KS0, jnp.uint32)
    x1 = c2 + ks[1]

    def rounds(x0, x1, rs):
        for r in rs:
            x0 = x0 + x1
            x1 = _rotl(x1, r)
            x1 = x0 ^ x1
        return x0, x1

    x0, x1 = rounds(x0, x1, _ROT[0])
    x0, x1 = x0 + ks[1], x1 + ks[2] + jnp.uint32(1)
    x0, x1 = rounds(x0, x1, _ROT[1])
    x0, x1 = x0 + ks[2], x1 + ks[0] + jnp.uint32(2)
    x0, x1 = rounds(x0, x1, _ROT[0])
    x0, x1 = x0 + ks[0], x1 + ks[1] + jnp.uint32(3)
    x0, x1 = rounds(x0, x1, _ROT[1])
    x0, x1 = x0 + ks[1], x1 + ks[2] + jnp.uint32(4)
    x0, x1 = rounds(x0, x1, _ROT[0])
    x0, x1 = x0 + ks[2], x1 + ks[0] + jnp.uint32(5)
    return x0 ^ x1


STRIP = 8


def _body(w_ref, out_ref, best_val, best_idx):
    j = pl.program_id(0)

    @pl.when(j == 0)
    def _():
        best_val[0] = -jnp.inf
        best_idx[0] = jnp.int32(2**31 - 1)

    row = jax.lax.broadcasted_iota(jnp.int32, (STRIP, COLS), 0)
    col = jax.lax.broadcasted_iota(jnp.int32, (STRIP, COLS), 1)
    pos0 = j * BLOCK_ROWS * COLS + row * COLS + col

    def step(i, carry):
        m_vec, idx_vec, pos = carry
        w = w_ref[pl.ds(i * STRIP, STRIP), :]
        bits = _threefry_bits(pos.astype(jnp.uint32))
        fbits = (bits >> jnp.uint32(9)) | jnp.uint32(0x3F800000)
        f = jax.lax.bitcast_convert_type(fbits, jnp.float32) - jnp.float32(1.0)
        u = jnp.maximum(f, jnp.float32(1.1754943508222875e-38))
        z = -jnp.log(-jnp.log(u)) + jnp.log(w)
        upd = z > m_vec
        m_vec = jnp.where(upd, z, m_vec)
        idx_vec = jnp.where(upd, pos, idx_vec)
        return m_vec, idx_vec, pos + STRIP * COLS

    m0 = jnp.full((STRIP, COLS), -jnp.inf, jnp.float32)
    i0 = jnp.full((STRIP, COLS), 2**31 - 1, jnp.int32)
    m_vec, idx_vec, _ = jax.lax.fori_loop(
        0, BLOCK_ROWS // STRIP, step, (m0, i0, pos0))

    m = jnp.max(m_vec)
    bi = jnp.min(jnp.where(m_vec == m, idx_vec, jnp.int32(2**31 - 1)))
    better = (m > best_val[0]) | ((m == best_val[0]) & (bi < best_idx[0]))
    best_val[0] = jnp.where(better, m, best_val[0])
    best_idx[0] = jnp.where(better, bi, best_idx[0])

    @pl.when(j == GRID - 1)
    def _():
        out_ref[0] = best_idx[0]


def kernel(weights):
    wp = jnp.pad(weights, (0, PAD - N)).reshape(ROWS, COLS)
    idx = pl.pallas_call(
        _body,
        grid=(GRID,),
        in_specs=[pl.BlockSpec((BLOCK_ROWS, COLS), lambda j: (j, 0))],
        out_specs=pl.BlockSpec(memory_space=pltpu.SMEM),
        out_shape=jax.ShapeDtypeStruct((1,), jnp.int32),
        scratch_shapes=[
            pltpu.SMEM((1,), jnp.float32),
            pltpu.SMEM((1,), jnp.int32),
        ],
    )(wp)
    return idx


# speedup vs baseline: 1.8112x; 1.1368x over previous
import jax
import jax.numpy as jnp
from jax.experimental import pallas as pl
from jax.experimental.pallas import tpu as pltpu

N = 1000000
ROWS, COLS = 1024, 1024
PAD = ROWS * COLS
BLOCK_ROWS = 128
GRID = ROWS // BLOCK_ROWS

_KS0 = 0
_KS1 = 42
_KS2 = _KS0 ^ _KS1 ^ 0x1BD11BDA
_ROT = ((13, 15, 26, 6), (17, 29, 16, 24))


def _rotl(x, d):
    return (x << jnp.uint32(d)) | (x >> jnp.uint32(32 - d))


def _threefry_bits(c2):
    ks = (jnp.uint32(_KS0), jnp.uint32(_KS1), jnp.uint32(_KS2))
    x0 = jnp.full(c2.shape, _KS0, jnp.uint32)
    x1 = c2 + ks[1]

    def rounds(x0, x1, rs):
        for r in rs:
            x0 = x0 + x1
            x1 = _rotl(x1, r)
            x1 = x0 ^ x1
        return x0, x1

    x0, x1 = rounds(x0, x1, _ROT[0])
    x0, x1 = x0 + ks[1], x1 + ks[2] + jnp.uint32(1)
    x0, x1 = rounds(x0, x1, _ROT[1])
    x0, x1 = x0 + ks[2], x1 + ks[0] + jnp.uint32(2)
    x0, x1 = rounds(x0, x1, _ROT[0])
    x0, x1 = x0 + ks[0], x1 + ks[1] + jnp.uint32(3)
    x0, x1 = rounds(x0, x1, _ROT[1])
    x0, x1 = x0 + ks[1], x1 + ks[2] + jnp.uint32(4)
    x0, x1 = rounds(x0, x1, _ROT[0])
    x0, x1 = x0 + ks[2], x1 + ks[0] + jnp.uint32(5)
    return x0 ^ x1


STRIP = 8


def _body(w_ref, out_ref, m_acc, idx_acc):
    j = pl.program_id(0)

    row = jax.lax.broadcasted_iota(jnp.int32, (STRIP, COLS), 0)
    col = jax.lax.broadcasted_iota(jnp.int32, (STRIP, COLS), 1)
    pos0 = j * BLOCK_ROWS * COLS + row * COLS + col

    def step(i, carry):
        m_vec, idx_vec, pos = carry
        w = w_ref[pl.ds(i * STRIP, STRIP), :]
        bits = _threefry_bits(pos.astype(jnp.uint32))
        fbits = (bits >> jnp.uint32(9)) | jnp.uint32(0x3F800000)
        f = jax.lax.bitcast_convert_type(fbits, jnp.float32) - jnp.float32(1.0)
        u = jnp.maximum(f, jnp.float32(1.1754943508222875e-38))
        z = -jnp.log(-jnp.log(u)) + jnp.log(w)
        upd = z > m_vec
        m_vec = jnp.where(upd, z, m_vec)
        idx_vec = jnp.where(upd, pos, idx_vec)
        return m_vec, idx_vec, pos + STRIP * COLS

    m0 = jnp.where(j == 0, jnp.full((STRIP, COLS), -jnp.inf, jnp.float32),
                   m_acc[...])
    i0 = jnp.where(j == 0, jnp.full((STRIP, COLS), 2**31 - 1, jnp.int32),
                   idx_acc[...])
    m_vec, idx_vec, _ = jax.lax.fori_loop(
        0, BLOCK_ROWS // STRIP, step, (m0, i0, pos0), unroll=2)
    m_acc[...] = m_vec
    idx_acc[...] = idx_vec

    @pl.when(j == GRID - 1)
    def _():
        m = jnp.max(m_vec)
        out_ref[0] = jnp.min(
            jnp.where(m_vec == m, idx_vec, jnp.int32(2**31 - 1)))


def kernel(weights):
    wp = jnp.pad(weights, (0, PAD - N)).reshape(ROWS, COLS)
    idx = pl.pallas_call(
        _body,
        grid=(GRID,),
        in_specs=[pl.BlockSpec((BLOCK_ROWS, COLS), lambda j: (j, 0))],
        out_specs=pl.BlockSpec(memory_space=pltpu.SMEM),
        out_shape=jax.ShapeDtypeStruct((1,), jnp.int32),
        scratch_shapes=[
            pltpu.VMEM((STRIP, COLS), jnp.float32),
            pltpu.VMEM((STRIP, COLS), jnp.int32),
        ],
    )(wp)
    return idx


# speedup vs baseline: 1.8237x; 1.0069x over previous
import jax
import jax.numpy as jnp
from jax.experimental import pallas as pl
from jax.experimental.pallas import tpu as pltpu

N = 1000000
ROWS, COLS = 1024, 1024
PAD = ROWS * COLS
BLOCK_ROWS = 128
GRID = ROWS // BLOCK_ROWS

_KS0 = 0
_KS1 = 42
_KS2 = _KS0 ^ _KS1 ^ 0x1BD11BDA
_ROT = ((13, 15, 26, 6), (17, 29, 16, 24))


def _rotl(x, d):
    return (x << jnp.uint32(d)) | (x >> jnp.uint32(32 - d))


def _threefry_bits(c2):
    ks = (jnp.uint32(_KS0), jnp.uint32(_KS1), jnp.uint32(_KS2))
    x0 = jnp.full(c2.shape, _KS0, jnp.uint32)
    x1 = c2 + ks[1]

    def rounds(x0, x1, rs):
        for r in rs:
            x0 = x0 + x1
            x1 = _rotl(x1, r)
            x1 = x0 ^ x1
        return x0, x1

    x0, x1 = rounds(x0, x1, _ROT[0])
    x0, x1 = x0 + ks[1], x1 + ks[2] + jnp.uint32(1)
    x0, x1 = rounds(x0, x1, _ROT[1])
    x0, x1 = x0 + ks[2], x1 + ks[0] + jnp.uint32(2)
    x0, x1 = rounds(x0, x1, _ROT[0])
    x0, x1 = x0 + ks[0], x1 + ks[1] + jnp.uint32(3)
    x0, x1 = rounds(x0, x1, _ROT[1])
    x0, x1 = x0 + ks[1], x1 + ks[2] + jnp.uint32(4)
    x0, x1 = rounds(x0, x1, _ROT[0])
    x0, x1 = x0 + ks[2], x1 + ks[0] + jnp.uint32(5)
    return x0 ^ x1


STRIP = 8


def _body(w_ref, out_ref, m_acc, idx_acc):
    j = pl.program_id(0)

    row = jax.lax.broadcasted_iota(jnp.int32, (STRIP, COLS), 0)
    col = jax.lax.broadcasted_iota(jnp.int32, (STRIP, COLS), 1)
    pos0 = j * BLOCK_ROWS * COLS + row * COLS + col

    def step(i, carry):
        m_vec, idx_vec = carry
        pos = pos0 + i * (STRIP * COLS)
        w = w_ref[pl.ds(i * STRIP, STRIP), :]
        bits = _threefry_bits(pos.astype(jnp.uint32))
        fbits = (bits >> jnp.uint32(9)) | jnp.uint32(0x3F800000)
        f = jax.lax.bitcast_convert_type(fbits, jnp.float32) - jnp.float32(1.0)
        u = jnp.maximum(f, jnp.float32(1.1754943508222875e-38))
        z = -jnp.log(-jnp.log(u)) + jnp.log(w)
        upd = z > m_vec
        m_vec = jnp.where(upd, z, m_vec)
        idx_vec = jnp.where(upd, pos, idx_vec)
        return m_vec, idx_vec

    m0 = jnp.where(j == 0, jnp.full((STRIP, COLS), -jnp.inf, jnp.float32),
                   m_acc[...])
    i0 = jnp.where(j == 0, jnp.full((STRIP, COLS), 2**31 - 1, jnp.int32),
                   idx_acc[...])
    m_vec, idx_vec = jax.lax.fori_loop(
        0, BLOCK_ROWS // STRIP, step, (m0, i0), unroll=4)
    m_acc[...] = m_vec
    idx_acc[...] = idx_vec

    @pl.when(j == GRID - 1)
    def _():
        m = jnp.max(m_vec)
        out_ref[0] = jnp.min(
            jnp.where(m_vec == m, idx_vec, jnp.int32(2**31 - 1)))


def kernel(weights):
    wp = jnp.pad(weights, (0, PAD - N)).reshape(ROWS, COLS)
    idx = pl.pallas_call(
        _body,
        grid=(GRID,),
        in_specs=[pl.BlockSpec((BLOCK_ROWS, COLS), lambda j: (j, 0))],
        out_specs=pl.BlockSpec(memory_space=pltpu.SMEM),
        out_shape=jax.ShapeDtypeStruct((1,), jnp.int32),
        scratch_shapes=[
            pltpu.VMEM((STRIP, COLS), jnp.float32),
            pltpu.VMEM((STRIP, COLS), jnp.int32),
        ],
    )(wp)
    return idx
